# baseline (speedup 1.0000x reference)
import functools

import numpy as np
import jax
import jax.numpy as jnp
from jax import lax
from jax.experimental import pallas as pl
from jax.experimental.pallas import tpu as pltpu

N_OBJ, N_DIM, N_SETS, N_NODE, B = 16, 1024, 8, 1024, 4096
BLK = 128
G = B // BLK + N_SETS
GB = G * BLK


def _f32_to_f64_words(o):
    bu = lax.bitcast_convert_type(o, jnp.uint32)
    exp8 = (bu >> 23) & jnp.uint32(0xFF)
    man = bu & jnp.uint32(0x7FFFFF)
    sign = bu & jnp.uint32(0x80000000)
    hi = sign | jnp.where(exp8 == 0, jnp.uint32(0),
                          ((exp8 + 896) << 20) | (man >> 3))
    hi = jnp.where(exp8 == 255, hi | jnp.uint32(0x7FF00000), hi)
    lo = jnp.where(exp8 == 0, jnp.uint32(0), man << 29)
    return (lax.bitcast_convert_type(lo, jnp.int32),
            lax.bitcast_convert_type(hi, jnp.int32))


def _mlp_body(bset_ref, x_ref, w1_ref, b1_ref, w2_ref, b2_ref, wh_ref,
              bh_ref, lo_ref, hi_ref):
    x = x_ref[...]
    h = jnp.dot(x, w1_ref[...], preferred_element_type=jnp.float32)
    h = jnp.maximum(h + b1_ref[...], 0.0)
    h = jnp.dot(h, w2_ref[...], preferred_element_type=jnp.float32)
    h = jnp.maximum(h + b2_ref[...], 0.0)
    o = lax.dot_general(h, wh_ref[0], (((1,), (1,)), ((), ())),
                        preferred_element_type=jnp.float32)
    o = jax.nn.sigmoid(o + bh_ref[0])
    lo, hi = _f32_to_f64_words(o)
    lo_ref[...] = lo
    hi_ref[...] = hi


_I0 = np.int32(0)

_grid_spec = pltpu.PrefetchScalarGridSpec(
    num_scalar_prefetch=1,
    grid=(G,),
    in_specs=[
        pl.BlockSpec((BLK, N_OBJ), lambda g, bset: (g, _I0)),
        pl.BlockSpec((N_OBJ, N_NODE), lambda g, bset: (_I0, _I0)),
        pl.BlockSpec((1, N_NODE), lambda g, bset: (_I0, _I0)),
        pl.BlockSpec((N_NODE, N_NODE), lambda g, bset: (_I0, _I0)),
        pl.BlockSpec((1, N_NODE), lambda g, bset: (_I0, _I0)),
        pl.BlockSpec((1, N_DIM, N_NODE), lambda g, bset: (bset[g], _I0, _I0)),
        pl.BlockSpec((1, 1, N_DIM), lambda g, bset: (bset[g], _I0, _I0)),
    ],
    out_specs=[pl.BlockSpec((BLK, N_DIM), lambda g, bset: (g, _I0)),
               pl.BlockSpec((BLK, N_DIM), lambda g, bset: (g, _I0))],
)

_mlp_call = pl.pallas_call(
    _mlp_body,
    grid_spec=_grid_spec,
    out_shape=(jax.ShapeDtypeStruct((GB, N_DIM), jnp.int32),
               jax.ShapeDtypeStruct((GB, N_DIM), jnp.int32)),
    compiler_params=pltpu.CompilerParams(
        dimension_semantics=("arbitrary",)),
)


@jax.jit
def kernel(pref, ps_id, W1, b1, W2, b2, Wh, bh):
    ps = ps_id.astype(jnp.int32)
    pref = pref.astype(jnp.float32)

    onehot = (ps[:, None] == jnp.arange(N_SETS, dtype=jnp.int32)[None, :]
              ).astype(jnp.int32)
    csum = jnp.cumsum(onehot, axis=0)
    counts = csum[-1]
    rank = jnp.sum(csum * onehot, axis=1) - 1

    nblk = (counts + BLK - 1) // BLK
    blk_cum0 = jnp.concatenate([jnp.zeros((1,), nblk.dtype),
                                jnp.cumsum(nblk)])
    my_base = jnp.sum(onehot * blk_cum0[None, :N_SETS], axis=1)
    slot = ((my_base + rank // BLK) * BLK + rank % BLK
            ).astype(jnp.int32)

    g = jnp.arange(G)
    bset = (jnp.sum(g[:, None] >= blk_cum0[None, 1:], axis=1)
            ).clip(0, N_SETS - 1).astype(jnp.int32)

    pref_sorted = jnp.zeros((GB, N_OBJ), jnp.float32).at[slot].set(pref)

    lo_sorted, hi_sorted = _mlp_call(
        bset, pref_sorted,
        W1.T.astype(jnp.float32),
        b1.reshape(1, N_NODE).astype(jnp.float32),
        W2.T.astype(jnp.float32),
        b2.reshape(1, N_NODE).astype(jnp.float32),
        Wh.astype(jnp.float32),
        bh.reshape(N_SETS, 1, N_DIM).astype(jnp.float32),
    )

    words = jnp.stack([lo_sorted[slot], hi_sorted[slot]], axis=-1)
    return lax.bitcast_convert_type(words, jnp.float64)


# speedup vs baseline: 1.2152x; 1.2152x over previous
import numpy as np
import jax
import jax.numpy as jnp
from jax import lax
from jax.experimental import pallas as pl
from jax.experimental.pallas import tpu as pltpu

N_OBJ, N_DIM, N_SETS, N_NODE, B = 16, 1024, 8, 1024, 4096
BLK = 128
G = B // BLK + N_SETS
GB = G * BLK


def _mlp_body(meta_ref, x_ref, w1_ref, b1_ref, w2_ref, b2_ref, wh_ref,
              bh_ref, o_ref):
    gid = pl.program_id(0)

    @pl.when(gid < meta_ref[G])
    def _():
        x = x_ref[...]
        h = jnp.dot(x, w1_ref[...], preferred_element_type=jnp.float32)
        h = jnp.maximum(h + b1_ref[...], 0.0)
        h = jnp.dot(h, w2_ref[...], preferred_element_type=jnp.float32)
        h = jnp.maximum(h + b2_ref[...], 0.0)
        o = lax.dot_general(h, wh_ref[0], (((1,), (1,)), ((), ())),
                            preferred_element_type=jnp.float32)
        o_ref[...] = jax.nn.sigmoid(o + bh_ref[0])


_I0 = np.int32(0)

_grid_spec = pltpu.PrefetchScalarGridSpec(
    num_scalar_prefetch=1,
    grid=(G,),
    in_specs=[
        pl.BlockSpec((BLK, N_OBJ), lambda g, meta: (g, _I0)),
        pl.BlockSpec((N_OBJ, N_NODE), lambda g, meta: (_I0, _I0)),
        pl.BlockSpec((1, N_NODE), lambda g, meta: (_I0, _I0)),
        pl.BlockSpec((N_NODE, N_NODE), lambda g, meta: (_I0, _I0)),
        pl.BlockSpec((1, N_NODE), lambda g, meta: (_I0, _I0)),
        pl.BlockSpec((1, N_DIM, N_NODE), lambda g, meta: (meta[g], _I0, _I0)),
        pl.BlockSpec((1, 1, N_DIM), lambda g, meta: (meta[g], _I0, _I0)),
    ],
    out_specs=pl.BlockSpec((BLK, N_DIM), lambda g, meta: (g, _I0)),
)

_mlp_call = pl.pallas_call(
    _mlp_body,
    grid_spec=_grid_spec,
    out_shape=jax.ShapeDtypeStruct((GB, N_DIM), jnp.float32),
    compiler_params=pltpu.CompilerParams(
        dimension_semantics=("arbitrary",)),
)


@jax.jit
def kernel(pref, ps_id, W1, b1, W2, b2, Wh, bh):
    ps = ps_id.astype(jnp.int32)
    pref = pref.astype(jnp.float32)

    onehot = (ps[:, None] == jnp.arange(N_SETS, dtype=jnp.int32)[None, :]
              ).astype(jnp.int32)
    csum = jnp.cumsum(onehot, axis=0)
    counts = csum[-1]
    rank = jnp.sum(csum * onehot, axis=1) - 1

    nblk = (counts + BLK - 1) // BLK
    blk_cum0 = jnp.concatenate([jnp.zeros((1,), nblk.dtype),
                                jnp.cumsum(nblk)])
    my_base = jnp.sum(onehot * blk_cum0[None, :N_SETS], axis=1)
    slot = ((my_base + rank // BLK) * BLK + rank % BLK
            ).astype(jnp.int32)

    g = jnp.arange(G)
    bset = (jnp.sum(g[:, None] >= blk_cum0[None, 1:], axis=1)
            ).clip(0, N_SETS - 1)
    meta = jnp.concatenate([bset, blk_cum0[-1:]]).astype(jnp.int32)

    pref_sorted = jnp.zeros((GB, N_OBJ), jnp.float32).at[slot].set(pref)

    out_sorted = _mlp_call(
        meta, pref_sorted,
        W1.T.astype(jnp.float32),
        b1.reshape(1, N_NODE).astype(jnp.float32),
        W2.T.astype(jnp.float32),
        b2.reshape(1, N_NODE).astype(jnp.float32),
        Wh.astype(jnp.float32),
        bh.reshape(N_SETS, 1, N_DIM).astype(jnp.float32),
    )

    return out_sorted[slot].astype(jnp.float64)


# speedup vs baseline: 1.2709x; 1.0458x over previous
import numpy as np
import jax
import jax.numpy as jnp
from jax import lax
from jax.experimental import pallas as pl
from jax.experimental.pallas import tpu as pltpu

N_OBJ, N_DIM, N_SETS, N_NODE, B = 16, 1024, 8, 1024, 4096
BLK = 256
G = B // BLK + N_SETS
GB = G * BLK


def _mlp_body(meta_ref, x_ref, w1_ref, b1_ref, w2_ref, b2_ref, wh_ref,
              bh_ref, o_ref):
    gid = pl.program_id(0)

    @pl.when(gid < meta_ref[G])
    def _():
        x = x_ref[...]
        h = jnp.dot(x, w1_ref[...], preferred_element_type=jnp.float32)
        h = jnp.maximum(h + b1_ref[...], 0.0)
        h = jnp.dot(h, w2_ref[...], preferred_element_type=jnp.float32)
        h = jnp.maximum(h + b2_ref[...], 0.0)
        o = lax.dot_general(h, wh_ref[0], (((1,), (1,)), ((), ())),
                            preferred_element_type=jnp.float32)
        o_ref[...] = jax.nn.sigmoid(o + bh_ref[0])


_I0 = np.int32(0)

_grid_spec = pltpu.PrefetchScalarGridSpec(
    num_scalar_prefetch=1,
    grid=(G,),
    in_specs=[
        pl.BlockSpec((BLK, N_OBJ), lambda g, meta: (g, _I0)),
        pl.BlockSpec((N_OBJ, N_NODE), lambda g, meta: (_I0, _I0)),
        pl.BlockSpec((1, N_NODE), lambda g, meta: (_I0, _I0)),
        pl.BlockSpec((N_NODE, N_NODE), lambda g, meta: (_I0, _I0)),
        pl.BlockSpec((1, N_NODE), lambda g, meta: (_I0, _I0)),
        pl.BlockSpec((1, N_DIM, N_NODE), lambda g, meta: (meta[g], _I0, _I0)),
        pl.BlockSpec((1, 1, N_DIM), lambda g, meta: (meta[g], _I0, _I0)),
    ],
    out_specs=pl.BlockSpec((BLK, N_DIM), lambda g, meta: (g, _I0)),
)

_mlp_call = pl.pallas_call(
    _mlp_body,
    grid_spec=_grid_spec,
    out_shape=jax.ShapeDtypeStruct((GB, N_DIM), jnp.float32),
    compiler_params=pltpu.CompilerParams(
        dimension_semantics=("arbitrary",)),
)


@jax.jit
def kernel(pref, ps_id, W1, b1, W2, b2, Wh, bh):
    ps = ps_id.astype(jnp.int32)
    pref = pref.astype(jnp.float32)

    onehot = (ps[:, None] == jnp.arange(N_SETS, dtype=jnp.int32)[None, :]
              ).astype(jnp.int32)
    csum = jnp.cumsum(onehot, axis=0)
    counts = csum[-1]
    rank = jnp.sum(csum * onehot, axis=1) - 1

    nblk = (counts + BLK - 1) // BLK
    blk_cum0 = jnp.concatenate([jnp.zeros((1,), nblk.dtype),
                                jnp.cumsum(nblk)])
    my_base = jnp.sum(onehot * blk_cum0[None, :N_SETS], axis=1)
    slot = ((my_base + rank // BLK) * BLK + rank % BLK
            ).astype(jnp.int32)

    g = jnp.arange(G)
    bset = (jnp.sum(g[:, None] >= blk_cum0[None, 1:], axis=1)
            ).clip(0, N_SETS - 1)
    meta = jnp.concatenate([bset, blk_cum0[-1:]]).astype(jnp.int32)

    pref_sorted = jnp.zeros((GB, N_OBJ), jnp.float32).at[slot].set(pref)

    out_sorted = _mlp_call(
        meta, pref_sorted,
        W1.T.astype(jnp.float32),
        b1.reshape(1, N_NODE).astype(jnp.float32),
        W2.T.astype(jnp.float32),
        b2.reshape(1, N_NODE).astype(jnp.float32),
        Wh.astype(jnp.float32),
        bh.reshape(N_SETS, 1, N_DIM).astype(jnp.float32),
    )

    return out_sorted[slot].astype(jnp.float64)


# speedup vs baseline: 1.2824x; 1.0090x over previous
import numpy as np
import jax
import jax.numpy as jnp
from jax import lax
from jax.experimental import pallas as pl
from jax.experimental.pallas import tpu as pltpu

N_OBJ, N_DIM, N_SETS, N_NODE, B = 16, 1024, 8, 1024, 4096
BLK = 512
G = B // BLK + N_SETS
GB = G * BLK


def _mlp_body(meta_ref, x_ref, w1_ref, b1_ref, w2_ref, b2_ref, wh_ref,
              bh_ref, o_ref):
    gid = pl.program_id(0)

    @pl.when(gid < meta_ref[G])
    def _():
        x = x_ref[...]
        h = jnp.dot(x, w1_ref[...], preferred_element_type=jnp.float32)
        h = jnp.maximum(h + b1_ref[...], 0.0)
        h = jnp.dot(h, w2_ref[...], preferred_element_type=jnp.float32)
        h = jnp.maximum(h + b2_ref[...], 0.0)
        o = lax.dot_general(h, wh_ref[0], (((1,), (1,)), ((), ())),
                            preferred_element_type=jnp.float32)
        o_ref[...] = jax.nn.sigmoid(o + bh_ref[0])


_I0 = np.int32(0)

_grid_spec = pltpu.PrefetchScalarGridSpec(
    num_scalar_prefetch=1,
    grid=(G,),
    in_specs=[
        pl.BlockSpec((BLK, N_OBJ), lambda g, meta: (g, _I0)),
        pl.BlockSpec((N_OBJ, N_NODE), lambda g, meta: (_I0, _I0)),
        pl.BlockSpec((1, N_NODE), lambda g, meta: (_I0, _I0)),
        pl.BlockSpec((N_NODE, N_NODE), lambda g, meta: (_I0, _I0)),
        pl.BlockSpec((1, N_NODE), lambda g, meta: (_I0, _I0)),
        pl.BlockSpec((1, N_DIM, N_NODE), lambda g, meta: (meta[g], _I0, _I0)),
        pl.BlockSpec((1, 1, N_DIM), lambda g, meta: (meta[g], _I0, _I0)),
    ],
    out_specs=pl.BlockSpec((BLK, N_DIM), lambda g, meta: (g, _I0)),
)

_mlp_call = pl.pallas_call(
    _mlp_body,
    grid_spec=_grid_spec,
    out_shape=jax.ShapeDtypeStruct((GB, N_DIM), jnp.float32),
    compiler_params=pltpu.CompilerParams(
        dimension_semantics=("arbitrary",)),
)


@jax.jit
def kernel(pref, ps_id, W1, b1, W2, b2, Wh, bh):
    ps = ps_id.astype(jnp.int32)
    pref = pref.astype(jnp.float32)

    onehot = (ps[:, None] == jnp.arange(N_SETS, dtype=jnp.int32)[None, :]
              ).astype(jnp.int32)
    csum = jnp.cumsum(onehot, axis=0)
    counts = csum[-1]
    rank = jnp.sum(csum * onehot, axis=1) - 1

    nblk = (counts + BLK - 1) // BLK
    blk_cum0 = jnp.concatenate([jnp.zeros((1,), nblk.dtype),
                                jnp.cumsum(nblk)])
    my_base = jnp.sum(onehot * blk_cum0[None, :N_SETS], axis=1)
    slot = ((my_base + rank // BLK) * BLK + rank % BLK
            ).astype(jnp.int32)

    g = jnp.arange(G)
    bset = (jnp.sum(g[:, None] >= blk_cum0[None, 1:], axis=1)
            ).clip(0, N_SETS - 1)
    meta = jnp.concatenate([bset, blk_cum0[-1:]]).astype(jnp.int32)

    pref_sorted = jnp.zeros((GB, N_OBJ), jnp.float32).at[slot].set(pref)

    out_sorted = _mlp_call(
        meta, pref_sorted,
        W1.T.astype(jnp.float32),
        b1.reshape(1, N_NODE).astype(jnp.float32),
        W2.T.astype(jnp.float32),
        b2.reshape(1, N_NODE).astype(jnp.float32),
        Wh.astype(jnp.float32),
        bh.reshape(N_SETS, 1, N_DIM).astype(jnp.float32),
    )

    return out_sorted[slot].astype(jnp.float64)
